# baseline (speedup 1.0000x reference)
import functools

import jax
import jax.numpy as jnp
from jax import lax
from jax.experimental import pallas as pl
from jax.experimental.pallas import tpu as pltpu
from jax.experimental.pallas import tpu_sc as plsc

N_NODES = 10000
N_EDGES = 320000
IN_CH = 128
OUT_CH = 10
S = 16
BOUND = 10
COMB = S * (BOUND + 1)
NL = 3

BLK = 2000
GRID = N_NODES // BLK

NC = 2
NS = 16
NW = NC * NS
EPT = N_EDGES // NW
EPAD = ((EPT + 127) // 128) * 128
NPC = (N_NODES * S) // NS
SH = N_NODES * S + 64


def _argmax_onehot_t(zt):
    m = jnp.max(zt, axis=0, keepdims=True)
    row = lax.broadcasted_iota(jnp.int32, zt.shape, 0)
    idx = jnp.min(jnp.where(zt >= m, row, S), axis=0)
    ht = (row == idx[None, :]).astype(jnp.float32)
    return ht, idx


def _in_body(x_ref, win_ref, bin_ref, wout_ref, bout_ref,
             h_ref, s_ref, loss_ref, m_ref):
    zt = lax.dot_general(win_ref[...], x_ref[...], (((0,), (1,)), ((), ())),
                         preferred_element_type=jnp.float32) + bin_ref[...]
    ht, idx = _argmax_onehot_t(zt)
    h_ref[...] = ht
    s_ref[...] = idx.reshape(1, 1, N_NODES)
    d = ht - zt
    loss_ref[0, 0] = jnp.sum(d * d) / (N_NODES * S)
    mm = wout_ref[...] + bout_ref[...]
    sh = mm - jnp.max(mm, axis=1, keepdims=True)
    m_ref[...] = sh - jnp.log(jnp.sum(jnp.exp(sh), axis=1, keepdims=True))


_in_call = pl.pallas_call(
    _in_body,
    out_specs=[
        pl.BlockSpec((S, N_NODES), lambda: (0, 0)),
        pl.BlockSpec((1, 1, N_NODES), lambda: (0, 0, 0)),
        pl.BlockSpec(memory_space=pltpu.SMEM),
        pl.BlockSpec((S, OUT_CH), lambda: (0, 0)),
    ],
    out_shape=[
        jax.ShapeDtypeStruct((S, N_NODES), jnp.float32),
        jax.ShapeDtypeStruct((1, 1, N_NODES), jnp.int32),
        jax.ShapeDtypeStruct((1, 1), jnp.float32),
        jax.ShapeDtypeStruct((S, OUT_CH), jnp.float32),
    ],
)


def _update_body(final, cnt_ref, h_ref, wlt_ref, bl_ref, m_ref,
                 ho_ref, so_ref, loss_ref, out_ref):
    cnt = cnt_ref[0] + cnt_ref[1]
    c = jnp.clip(cnt, 0.0, float(BOUND))
    rrow = lax.broadcasted_iota(jnp.int32, (S * BOUND, S), 0)
    rcol = lax.broadcasted_iota(jnp.int32, (S * BOUND, S), 1)
    rep = (rrow // BOUND == rcol).astype(jnp.float32)
    cexp = jnp.dot(rep, c, preferred_element_type=jnp.float32)
    y = (lax.broadcasted_iota(jnp.int32, (S * BOUND, N_NODES), 0) % BOUND
         ).astype(jnp.float32)
    t = cexp - y
    el = jnp.where(t > 0, t, jnp.exp(t) - 1.0)
    st = jax.nn.sigmoid(el - 0.5)
    zt = (jnp.dot(wlt_ref[:, :S * BOUND], st,
                  preferred_element_type=jnp.float32)
          + jnp.dot(wlt_ref[:, S * BOUND:], h_ref[...],
                    preferred_element_type=jnp.float32)
          + bl_ref[...])
    ht, idx = _argmax_onehot_t(zt)
    ho_ref[...] = ht
    so_ref[...] = idx.reshape(1, 1, N_NODES)
    if final:
        out_ref[...] = lax.dot_general(ht, m_ref[...], (((0,), (0,)), ((), ())),
                                       preferred_element_type=jnp.float32)
    d = ht - zt
    loss_ref[0, 0] = jnp.sum(d * d) / (N_NODES * S)


def _make_update(final):
    out_specs = [
        pl.BlockSpec((S, N_NODES), lambda: (0, 0)),
        pl.BlockSpec((1, 1, N_NODES), lambda: (0, 0, 0)),
        pl.BlockSpec(memory_space=pltpu.SMEM),
    ]
    out_shape = [
        jax.ShapeDtypeStruct((S, N_NODES), jnp.float32),
        jax.ShapeDtypeStruct((1, 1, N_NODES), jnp.int32),
        jax.ShapeDtypeStruct((1, 1), jnp.float32),
    ]
    if final:
        out_specs.append(pl.BlockSpec((N_NODES, OUT_CH), lambda: (0, 0)))
        out_shape.append(jax.ShapeDtypeStruct((N_NODES, OUT_CH), jnp.float32))
    else:
        out_specs.append(pl.BlockSpec((1, 1), lambda: (0, 0)))
        out_shape.append(jax.ShapeDtypeStruct((1, 1), jnp.float32))
    return pl.pallas_call(
        functools.partial(_update_body, final),
        out_specs=out_specs,
        out_shape=out_shape,
    )


_update_call = _make_update(False)
_update_final_call = _make_update(True)


EBASE = 9984
EPAD = N_EDGES - (NW - 1) * EBASE


def _count_body(s_hbm, ei_hbm, out_hbm,
                s_v, ed_v, idx_v, ones_v, zbuf, cnt_sh, sem_s, sem_e, sem_z):
    cid = lax.axis_index("c")
    sid = lax.axis_index("s")
    wid = cid * NS + sid
    base = wid * EBASE
    ngreal = jnp.where(wid == NW - 1, EPAD // 16, EBASE // 16)

    cs = pltpu.async_copy(s_hbm.at[0, 0], s_v, sem_s)
    ce = pltpu.async_copy(ei_hbm.at[:, pl.ds(base, EPAD)], ed_v, sem_e)

    def zb(i, _):
        zbuf[pl.ds(i * 16, 16)] = jnp.zeros((16,), jnp.float32)
        return 0
    lax.fori_loop(0, NPC // 16, zb, 0, unroll=8)
    cz = pltpu.async_copy(zbuf, cnt_sh.at[pl.ds(sid * NPC, NPC)], sem_z)

    def ob(i, _):
        ones_v[pl.ds(i * 16, 16)] = jnp.ones((16,), jnp.float32)
        return 0
    lax.fori_loop(0, EPAD // 16, ob, 0, unroll=8)

    with jax.named_scope("ph_wait_in"):
        cs.wait()
        ce.wait()

    with jax.named_scope("ph_gather"):
        def gb(i, _):
            sv = ed_v[0, pl.ds(i * 16, 16)]
            dv = ed_v[1, pl.ds(i * 16, 16)]
            j = plsc.load_gather(s_v, [sv])
            idx_v[pl.ds(i * 16, 16)] = j * N_NODES + dv
            return 0
        lax.fori_loop(0, EPAD // 16, gb, 0, unroll=8)

        def pb(i, _):
            idx_v[pl.ds(i * 16, 16)] = jnp.full((16,), N_NODES * S, jnp.int32)
            return 0
        lax.fori_loop(ngreal, EPAD // 16, pb, 0)

    with jax.named_scope("ph_zwait"):
        cz.wait()
        plsc.subcore_barrier()
    with jax.named_scope("ph_scatter"):
        pltpu.sync_copy(ones_v, cnt_sh.at[idx_v], add=True)
    with jax.named_scope("ph_bar2"):
        plsc.subcore_barrier()
    with jax.named_scope("ph_out"):
        pltpu.sync_copy(cnt_sh.at[pl.ds(sid * NPC, NPC)], zbuf)
        pltpu.sync_copy(zbuf,
                        out_hbm.at[pl.ds(cid * N_NODES * S + sid * NPC, NPC)])


@functools.cache
def _get_count_call():
  return functools.partial(
    pl.kernel,
    out_type=jax.ShapeDtypeStruct((NC * N_NODES * S,), jnp.float32),
    mesh=plsc.VectorSubcoreMesh(core_axis_name="c", subcore_axis_name="s"),
    compiler_params=pltpu.CompilerParams(needs_layout_passes=False),
    scratch_types=[
        pltpu.VMEM((N_NODES,), jnp.int32),
        pltpu.VMEM((2, EPAD), jnp.int32),
        pltpu.VMEM((EPAD,), jnp.int32),
        pltpu.VMEM((EPAD,), jnp.float32),
        pltpu.VMEM((NPC,), jnp.float32),
        pltpu.VMEM_SHARED((SH,), jnp.float32),
        pltpu.SemaphoreType.DMA,
        pltpu.SemaphoreType.DMA,
        pltpu.SemaphoreType.DMA,
    ],
  )(_count_body)


def kernel(x, edge_index, W_in, b_in, W_layers, b_layers, W_out, b_out):
    W_layers_t = jnp.swapaxes(W_layers, 1, 2)
    h, s3, loss0, m = _in_call(x.astype(jnp.float32), W_in,
                               b_in.reshape(S, 1), W_out,
                               b_out.reshape(1, OUT_CH))
    losses = [loss0[0, 0]]
    for l in range(NL):
        cnt2 = _get_count_call()(s3, edge_index)
        cnt2 = cnt2.reshape(NC, S, N_NODES)
        call = _update_final_call if l == NL - 1 else _update_call
        h, s3, ll, out = call(cnt2, h, W_layers_t[l],
                              b_layers[l].reshape(S, 1), m)
        losses.append(ll[0, 0])
    return (out, jnp.stack(losses))


# speedup vs baseline: 1.1608x; 1.1608x over previous
import functools

import jax
import jax.numpy as jnp
from jax import lax
from jax.experimental import pallas as pl
from jax.experimental.pallas import tpu as pltpu
from jax.experimental.pallas import tpu_sc as plsc

N_NODES = 10000
N_EDGES = 320000
IN_CH = 128
OUT_CH = 10
S = 16
BOUND = 10
COMB = S * (BOUND + 1)
NL = 3

BLK = 2000
GRID = N_NODES // BLK

NC = 2
NS = 16
NW = NC * NS
EPT = N_EDGES // NW
EPAD = ((EPT + 127) // 128) * 128
NPC = (N_NODES * S) // NS
SH = N_NODES * S + 64


def _argmax_onehot_t(zt):
    m = jnp.max(zt, axis=0, keepdims=True)
    row = lax.broadcasted_iota(jnp.int32, zt.shape, 0)
    idx = jnp.min(jnp.where(zt >= m, row, S), axis=0)
    ht = (row == idx[None, :]).astype(jnp.float32)
    return ht, idx


def _in_body(x_ref, win_ref, bin_ref, wout_ref, bout_ref,
             h_ref, s_ref, loss_ref, m_ref):
    zt = lax.dot_general(win_ref[...], x_ref[...], (((0,), (1,)), ((), ())),
                         preferred_element_type=jnp.float32) + bin_ref[...]
    ht, idx = _argmax_onehot_t(zt)
    h_ref[...] = ht
    s_ref[...] = idx.reshape(1, 1, N_NODES)
    d = ht - zt
    loss_ref[0, 0] = jnp.sum(d * d) / (N_NODES * S)
    mm = wout_ref[...] + bout_ref[...]
    sh = mm - jnp.max(mm, axis=1, keepdims=True)
    m_ref[...] = sh - jnp.log(jnp.sum(jnp.exp(sh), axis=1, keepdims=True))


_in_call = pl.pallas_call(
    _in_body,
    out_specs=[
        pl.BlockSpec((S, N_NODES), lambda: (0, 0)),
        pl.BlockSpec((1, 1, N_NODES), lambda: (0, 0, 0)),
        pl.BlockSpec(memory_space=pltpu.SMEM),
        pl.BlockSpec((S, OUT_CH), lambda: (0, 0)),
    ],
    out_shape=[
        jax.ShapeDtypeStruct((S, N_NODES), jnp.float32),
        jax.ShapeDtypeStruct((1, 1, N_NODES), jnp.int32),
        jax.ShapeDtypeStruct((1, 1), jnp.float32),
        jax.ShapeDtypeStruct((S, OUT_CH), jnp.float32),
    ],
)


def _update_body(final, cnt_ref, h_ref, wlt_ref, bl_ref, m_ref,
                 ho_ref, so_ref, loss_ref, out_ref):
    cnt = cnt_ref[0] + cnt_ref[1]
    c = jnp.clip(cnt, 0.0, float(BOUND))
    rrow = lax.broadcasted_iota(jnp.int32, (S * BOUND, S), 0)
    rcol = lax.broadcasted_iota(jnp.int32, (S * BOUND, S), 1)
    rep = (rrow // BOUND == rcol).astype(jnp.float32)
    cexp = jnp.dot(rep, c, preferred_element_type=jnp.float32)
    y = (lax.broadcasted_iota(jnp.int32, (S * BOUND, N_NODES), 0) % BOUND
         ).astype(jnp.float32)
    t = cexp - y
    el = jnp.where(t > 0, t, jnp.exp(t) - 1.0)
    st = jax.nn.sigmoid(el - 0.5)
    zt = (jnp.dot(wlt_ref[:, :S * BOUND], st,
                  preferred_element_type=jnp.float32)
          + jnp.dot(wlt_ref[:, S * BOUND:], h_ref[...],
                    preferred_element_type=jnp.float32)
          + bl_ref[...])
    ht, idx = _argmax_onehot_t(zt)
    ho_ref[...] = ht
    so_ref[...] = idx.reshape(1, 1, N_NODES)
    if final:
        out_ref[...] = lax.dot_general(ht, m_ref[...], (((0,), (0,)), ((), ())),
                                       preferred_element_type=jnp.float32)
    d = ht - zt
    loss_ref[0, 0] = jnp.sum(d * d) / (N_NODES * S)


def _make_update(final):
    out_specs = [
        pl.BlockSpec((S, N_NODES), lambda: (0, 0)),
        pl.BlockSpec((1, 1, N_NODES), lambda: (0, 0, 0)),
        pl.BlockSpec(memory_space=pltpu.SMEM),
    ]
    out_shape = [
        jax.ShapeDtypeStruct((S, N_NODES), jnp.float32),
        jax.ShapeDtypeStruct((1, 1, N_NODES), jnp.int32),
        jax.ShapeDtypeStruct((1, 1), jnp.float32),
    ]
    if final:
        out_specs.append(pl.BlockSpec((N_NODES, OUT_CH), lambda: (0, 0)))
        out_shape.append(jax.ShapeDtypeStruct((N_NODES, OUT_CH), jnp.float32))
    else:
        out_specs.append(pl.BlockSpec((1, 1), lambda: (0, 0)))
        out_shape.append(jax.ShapeDtypeStruct((1, 1), jnp.float32))
    return pl.pallas_call(
        functools.partial(_update_body, final),
        out_specs=out_specs,
        out_shape=out_shape,
    )


_update_call = _make_update(False)
_update_final_call = _make_update(True)


EBASE = 9984
EPAD = N_EDGES - (NW - 1) * EBASE


HEP = EPAD // 2


def _count_body(s_hbm, ei_hbm, out_hbm,
                s_v, ed_v, idx_a, idx_b, ones_v, zbuf, cnt_sh,
                sem_s, sem_e, sem_z, sem_a, sem_b):
    cid = lax.axis_index("c")
    sid = lax.axis_index("s")
    wid = cid * NS + sid
    base = wid * EBASE
    ngreal = jnp.where(wid == NW - 1, EPAD // 16, EBASE // 16)

    cs = pltpu.async_copy(s_hbm.at[0, 0], s_v, sem_s)
    ce = pltpu.async_copy(ei_hbm.at[:, pl.ds(base, EPAD)], ed_v, sem_e)

    def zb(i, _):
        zbuf[pl.ds(i * 16, 16)] = jnp.zeros((16,), jnp.float32)
        return 0
    lax.fori_loop(0, NPC // 16, zb, 0, unroll=8)
    cz = pltpu.async_copy(zbuf, cnt_sh.at[pl.ds(sid * NPC, NPC)], sem_z)

    def ob(i, _):
        ones_v[pl.ds(i * 16, 16)] = jnp.ones((16,), jnp.float32)
        return 0
    lax.fori_loop(0, HEP // 16, ob, 0, unroll=8)

    with jax.named_scope("ph_wait_in"):
        cs.wait()
        ce.wait()

    with jax.named_scope("ph_gather_a"):
        @plsc.parallel_loop(0, HEP // 16, unroll=8)
        def ga(i):
            sv = ed_v[0, pl.ds(i * 16, 16)]
            dv = ed_v[1, pl.ds(i * 16, 16)]
            j = plsc.load_gather(s_v, [sv])
            idx_a[pl.ds(i * 16, 16)] = j * N_NODES + dv

    with jax.named_scope("ph_zwait"):
        cz.wait()
        plsc.subcore_barrier()
    ca = pltpu.async_copy(ones_v, cnt_sh.at[idx_a], sem_a, add=True)

    with jax.named_scope("ph_gather_b"):
        @plsc.parallel_loop(0, HEP // 16, unroll=8)
        def gb(i):
            sv = ed_v[0, pl.ds(HEP + i * 16, 16)]
            dv = ed_v[1, pl.ds(HEP + i * 16, 16)]
            j = plsc.load_gather(s_v, [sv])
            idx_b[pl.ds(i * 16, 16)] = j * N_NODES + dv

        def pb(i, _):
            idx_b[pl.ds(i * 16, 16)] = jnp.full((16,), N_NODES * S, jnp.int32)
            return 0
        lax.fori_loop(jnp.maximum(ngreal - HEP // 16, 0), HEP // 16, pb, 0)

    cb = pltpu.async_copy(ones_v, cnt_sh.at[idx_b], sem_b, add=True)
    with jax.named_scope("ph_scatter"):
        ca.wait()
        cb.wait()
    with jax.named_scope("ph_bar2"):
        plsc.subcore_barrier()
    with jax.named_scope("ph_out"):
        pltpu.sync_copy(cnt_sh.at[pl.ds(sid * NPC, NPC)], zbuf)
        pltpu.sync_copy(zbuf,
                        out_hbm.at[pl.ds(cid * N_NODES * S + sid * NPC, NPC)])


@functools.cache
def _get_count_call():
  return functools.partial(
    pl.kernel,
    out_type=jax.ShapeDtypeStruct((NC * N_NODES * S,), jnp.float32),
    mesh=plsc.VectorSubcoreMesh(core_axis_name="c", subcore_axis_name="s"),
    compiler_params=pltpu.CompilerParams(needs_layout_passes=False),
    scratch_types=[
        pltpu.VMEM((N_NODES,), jnp.int32),
        pltpu.VMEM((2, EPAD), jnp.int32),
        pltpu.VMEM((HEP,), jnp.int32),
        pltpu.VMEM((HEP,), jnp.int32),
        pltpu.VMEM((HEP,), jnp.float32),
        pltpu.VMEM((NPC,), jnp.float32),
        pltpu.VMEM_SHARED((SH,), jnp.float32),
        pltpu.SemaphoreType.DMA,
        pltpu.SemaphoreType.DMA,
        pltpu.SemaphoreType.DMA,
        pltpu.SemaphoreType.DMA,
        pltpu.SemaphoreType.DMA,
    ],
  )(_count_body)


def kernel(x, edge_index, W_in, b_in, W_layers, b_layers, W_out, b_out):
    W_layers_t = jnp.swapaxes(W_layers, 1, 2)
    h, s3, loss0, m = _in_call(x.astype(jnp.float32), W_in,
                               b_in.reshape(S, 1), W_out,
                               b_out.reshape(1, OUT_CH))
    losses = [loss0[0, 0]]
    for l in range(NL):
        cnt2 = _get_count_call()(s3, edge_index)
        cnt2 = cnt2.reshape(NC, S, N_NODES)
        call = _update_final_call if l == NL - 1 else _update_call
        h, s3, ll, out = call(cnt2, h, W_layers_t[l],
                              b_layers[l].reshape(S, 1), m)
        losses.append(ll[0, 0])
    return (out, jnp.stack(losses))


# speedup vs baseline: 1.2240x; 1.0544x over previous
import functools

import jax
import jax.numpy as jnp
from jax import lax
from jax.experimental import pallas as pl
from jax.experimental.pallas import tpu as pltpu
from jax.experimental.pallas import tpu_sc as plsc

N_NODES = 10000
N_EDGES = 320000
IN_CH = 128
OUT_CH = 10
S = 16
BOUND = 10
COMB = S * (BOUND + 1)
NL = 3

BLK = 2000
GRID = N_NODES // BLK

NC = 2
NS = 16
NW = NC * NS
EPT = N_EDGES // NW
EPAD = ((EPT + 127) // 128) * 128
NPC = (N_NODES * S) // NS
SH = N_NODES * S + 64


def _argmax_onehot_t(zt):
    m = jnp.max(zt, axis=0, keepdims=True)
    row = lax.broadcasted_iota(jnp.int32, zt.shape, 0)
    idx = jnp.min(jnp.where(zt >= m, row, S), axis=0)
    ht = (row == idx[None, :]).astype(jnp.float32)
    return ht, idx


def _in_body(x_ref, win_ref, bin_ref, wout_ref, bout_ref,
             h_ref, s_ref, loss_ref, m_ref):
    zt = lax.dot_general(win_ref[...], x_ref[...], (((0,), (1,)), ((), ())),
                         preferred_element_type=jnp.float32) + bin_ref[...]
    ht, idx = _argmax_onehot_t(zt)
    h_ref[...] = ht
    s_ref[...] = idx.reshape(1, 1, N_NODES)
    d = ht - zt
    loss_ref[0, 0] = jnp.sum(d * d) / (N_NODES * S)
    mm = wout_ref[...] + bout_ref[...]
    sh = mm - jnp.max(mm, axis=1, keepdims=True)
    m_ref[...] = sh - jnp.log(jnp.sum(jnp.exp(sh), axis=1, keepdims=True))


_in_call = pl.pallas_call(
    _in_body,
    out_specs=[
        pl.BlockSpec((S, N_NODES), lambda: (0, 0)),
        pl.BlockSpec((1, 1, N_NODES), lambda: (0, 0, 0)),
        pl.BlockSpec(memory_space=pltpu.SMEM),
        pl.BlockSpec((S, OUT_CH), lambda: (0, 0)),
    ],
    out_shape=[
        jax.ShapeDtypeStruct((S, N_NODES), jnp.float32),
        jax.ShapeDtypeStruct((1, 1, N_NODES), jnp.int32),
        jax.ShapeDtypeStruct((1, 1), jnp.float32),
        jax.ShapeDtypeStruct((S, OUT_CH), jnp.float32),
    ],
)


def _update_body(final, cnt_ref, h_ref, wlt_ref, bl_ref, m_ref,
                 ho_ref, so_ref, loss_ref, out_ref):
    cnt = cnt_ref[0] + cnt_ref[1]
    c = jnp.clip(cnt, 0.0, float(BOUND))
    rrow = lax.broadcasted_iota(jnp.int32, (S * BOUND, S), 0)
    rcol = lax.broadcasted_iota(jnp.int32, (S * BOUND, S), 1)
    rep = (rrow // BOUND == rcol).astype(jnp.float32)
    cexp = jnp.dot(rep, c, preferred_element_type=jnp.float32)
    y = (lax.broadcasted_iota(jnp.int32, (S * BOUND, N_NODES), 0) % BOUND
         ).astype(jnp.float32)
    t = cexp - y
    el = jnp.where(t > 0, t, jnp.exp(t) - 1.0)
    st = jax.nn.sigmoid(el - 0.5)
    zt = (jnp.dot(wlt_ref[:, :S * BOUND], st,
                  preferred_element_type=jnp.float32)
          + jnp.dot(wlt_ref[:, S * BOUND:], h_ref[...],
                    preferred_element_type=jnp.float32)
          + bl_ref[...])
    ht, idx = _argmax_onehot_t(zt)
    ho_ref[...] = ht
    so_ref[...] = idx.reshape(1, 1, N_NODES)
    if final:
        out_ref[...] = lax.dot_general(m_ref[...], ht, (((0,), (0,)), ((), ())),
                                       preferred_element_type=jnp.float32)
    d = ht - zt
    loss_ref[0, 0] = jnp.sum(d * d) / (N_NODES * S)


def _make_update(final):
    out_specs = [
        pl.BlockSpec((S, N_NODES), lambda: (0, 0)),
        pl.BlockSpec((1, 1, N_NODES), lambda: (0, 0, 0)),
        pl.BlockSpec(memory_space=pltpu.SMEM),
    ]
    out_shape = [
        jax.ShapeDtypeStruct((S, N_NODES), jnp.float32),
        jax.ShapeDtypeStruct((1, 1, N_NODES), jnp.int32),
        jax.ShapeDtypeStruct((1, 1), jnp.float32),
    ]
    if final:
        out_specs.append(pl.BlockSpec((OUT_CH, N_NODES), lambda: (0, 0)))
        out_shape.append(jax.ShapeDtypeStruct((OUT_CH, N_NODES), jnp.float32))
    else:
        out_specs.append(pl.BlockSpec((1, 1), lambda: (0, 0)))
        out_shape.append(jax.ShapeDtypeStruct((1, 1), jnp.float32))
    return pl.pallas_call(
        functools.partial(_update_body, final),
        out_specs=out_specs,
        out_shape=out_shape,
    )


_update_call = _make_update(False)
_update_final_call = _make_update(True)


EBASE = 9984
EPAD = N_EDGES - (NW - 1) * EBASE


HEP = EPAD // 2


def _count_body(s_hbm, ei_hbm, out_hbm,
                s_v, ed_v, idx_a, idx_b, ones_v, zbuf, cnt_sh,
                sem_s, sem_e, sem_z, sem_a, sem_b, sem_b2):
    cid = lax.axis_index("c")
    sid = lax.axis_index("s")
    wid = cid * NS + sid
    base = wid * EBASE
    ngreal = jnp.where(wid == NW - 1, EPAD // 16, EBASE // 16)

    cs = pltpu.async_copy(s_hbm.at[0, 0], s_v, sem_s)
    ce_a = pltpu.async_copy(ei_hbm.at[:, pl.ds(base, HEP)],
                            ed_v.at[:, pl.ds(0, HEP)], sem_e)
    ce_b = pltpu.async_copy(ei_hbm.at[:, pl.ds(base + HEP, HEP)],
                            ed_v.at[:, pl.ds(HEP, HEP)], sem_b2)

    def zb(i, _):
        zbuf[pl.ds(i * 16, 16)] = jnp.zeros((16,), jnp.float32)
        return 0
    lax.fori_loop(0, NPC // 16, zb, 0, unroll=8)
    cz = pltpu.async_copy(zbuf, cnt_sh.at[pl.ds(sid * NPC, NPC)], sem_z)

    def ob(i, _):
        ones_v[pl.ds(i * 16, 16)] = jnp.ones((16,), jnp.float32)
        return 0
    lax.fori_loop(0, HEP // 16, ob, 0, unroll=8)

    with jax.named_scope("ph_wait_in"):
        cs.wait()
        ce_a.wait()

    with jax.named_scope("ph_gather_a"):
        @plsc.parallel_loop(0, HEP // 16, unroll=8)
        def ga(i):
            sv = ed_v[0, pl.ds(i * 16, 16)]
            dv = ed_v[1, pl.ds(i * 16, 16)]
            j = plsc.load_gather(s_v, [sv])
            idx_a[pl.ds(i * 16, 16)] = j * N_NODES + dv

    with jax.named_scope("ph_zwait"):
        cz.wait()
        plsc.subcore_barrier()
    ca = pltpu.async_copy(ones_v, cnt_sh.at[idx_a], sem_a, add=True)

    with jax.named_scope("ph_wait_in2"):
        ce_b.wait()
    with jax.named_scope("ph_gather_b"):
        @plsc.parallel_loop(0, HEP // 16, unroll=8)
        def gb(i):
            sv = ed_v[0, pl.ds(HEP + i * 16, 16)]
            dv = ed_v[1, pl.ds(HEP + i * 16, 16)]
            j = plsc.load_gather(s_v, [sv])
            idx_b[pl.ds(i * 16, 16)] = j * N_NODES + dv

        def pb(i, _):
            idx_b[pl.ds(i * 16, 16)] = jnp.full((16,), N_NODES * S, jnp.int32)
            return 0
        lax.fori_loop(jnp.maximum(ngreal - HEP // 16, 0), HEP // 16, pb, 0)

    cb = pltpu.async_copy(ones_v, cnt_sh.at[idx_b], sem_b, add=True)
    with jax.named_scope("ph_scatter"):
        ca.wait()
        cb.wait()
    with jax.named_scope("ph_bar2"):
        plsc.subcore_barrier()
    with jax.named_scope("ph_out"):
        pltpu.sync_copy(cnt_sh.at[pl.ds(sid * NPC, NPC)], zbuf)
        pltpu.sync_copy(zbuf,
                        out_hbm.at[pl.ds(cid * N_NODES * S + sid * NPC, NPC)])


@functools.cache
def _get_count_call():
  return functools.partial(
    pl.kernel,
    out_type=jax.ShapeDtypeStruct((NC * N_NODES * S,), jnp.float32),
    mesh=plsc.VectorSubcoreMesh(core_axis_name="c", subcore_axis_name="s"),
    compiler_params=pltpu.CompilerParams(needs_layout_passes=False),
    scratch_types=[
        pltpu.VMEM((N_NODES,), jnp.int32),
        pltpu.VMEM((2, EPAD), jnp.int32),
        pltpu.VMEM((HEP,), jnp.int32),
        pltpu.VMEM((HEP,), jnp.int32),
        pltpu.VMEM((HEP,), jnp.float32),
        pltpu.VMEM((NPC,), jnp.float32),
        pltpu.VMEM_SHARED((SH,), jnp.float32),
        pltpu.SemaphoreType.DMA,
        pltpu.SemaphoreType.DMA,
        pltpu.SemaphoreType.DMA,
        pltpu.SemaphoreType.DMA,
        pltpu.SemaphoreType.DMA,
        pltpu.SemaphoreType.DMA,
    ],
  )(_count_body)


def kernel(x, edge_index, W_in, b_in, W_layers, b_layers, W_out, b_out):
    W_layers_t = jnp.swapaxes(W_layers, 1, 2)
    h, s3, loss0, m = _in_call(x.astype(jnp.float32), W_in,
                               b_in.reshape(S, 1), W_out,
                               b_out.reshape(1, OUT_CH))
    losses = [loss0[0, 0]]
    for l in range(NL):
        cnt2 = _get_count_call()(s3, edge_index)
        cnt2 = cnt2.reshape(NC, S, N_NODES)
        call = _update_final_call if l == NL - 1 else _update_call
        h, s3, ll, out = call(cnt2, h, W_layers_t[l],
                              b_layers[l].reshape(S, 1), m)
        losses.append(ll[0, 0])
    return (out.T, jnp.stack(losses))
